# baseline (speedup 1.0000x reference)
import functools

import jax
import jax.numpy as jnp
from jax import lax
from jax.experimental import pallas as pl
from jax.experimental.pallas import tpu as pltpu
from jax.experimental.pallas import tpu_sc as plsc

_B, _S, _D = 4, 2048, 768
_L = 16
_NC, _NS = 2, 16
_NW = _NC * _NS
_RPW = (_B * _S) // _NW
_R = 16
_NCHUNK = _RPW // _R
_NDC = _D // _L
_SPW = _S // _RPW


def _allsum_vec(x):
    for k in (8, 4, 2, 1):
        perm = lax.bitwise_xor(lax.iota(jnp.int32, _L), jnp.int32(k))
        x = x + jnp.take(x, perm)
    return x


def _rsqrt_vec(x):
    one = jnp.float32(1.0)
    s = jnp.full((_L,), one)
    r = jnp.full((_L,), one)
    for _ in range(10):
        xs = x * s
        big = xs >= jnp.float32(2.0)
        small = xs < jnp.float32(0.5)
        s = jnp.where(big, s * jnp.float32(0.25), s)
        r = jnp.where(big, r * jnp.float32(0.5), r)
        s = jnp.where(small, s * jnp.float32(4.0), s)
        r = jnp.where(small, r * jnp.float32(2.0), r)
    z = x * s
    y = jnp.float32(1.788) - z * (jnp.float32(0.813)
                                  - z * jnp.float32(0.153))
    for _ in range(3):
        y = y * (jnp.float32(1.5) - jnp.float32(0.5) * z * y * y)
    return y * r


def _emb_body(tok_idx, seg_idx, table, seg_table, pos, gamma, beta, out,
              idx_v, sidx_v, rows_v, pos_v, seg_v, gamma_v, beta_v, sem):
    wid = lax.axis_index("s") * _NC + lax.axis_index("c")
    pltpu.sync_copy(tok_idx.at[wid], idx_v)
    pltpu.sync_copy(seg_idx.at[wid], sidx_v)
    pltpu.sync_copy(gamma, gamma_v)
    pltpu.sync_copy(beta, beta_v)
    s0 = (wid % _SPW) * _RPW

    def chunk_body(ci, carry):
        g0 = wid * _RPW + ci * _R
        sc0 = s0 + ci * _R
        cp1 = pltpu.async_copy(table.at[idx_v.at[ci]], rows_v, sem)
        cp2 = pltpu.async_copy(seg_table.at[sidx_v.at[ci]], seg_v, sem)
        cp3 = pltpu.async_copy(pos.at[pl.ds(sc0, _R)], pos_v, sem)
        cp1.wait()
        cp2.wait()
        cp3.wait()

        def row_body(r, rcarry):
            acc = jnp.zeros((_L,), jnp.float32)
            acc2 = jnp.zeros((_L,), jnp.float32)
            for c in range(_NDC):
                sl = pl.ds(c * _L, _L)
                x = rows_v[r, sl] + seg_v[r, sl] + pos_v[r, sl]
                rows_v[r, sl] = x
                acc = acc + x
                acc2 = acc2 + x * x
            mu = _allsum_vec(acc) * jnp.float32(1.0 / _D)
            var = _allsum_vec(acc2) * jnp.float32(1.0 / _D) - mu * mu
            inv = _rsqrt_vec(var + jnp.float32(1e-5))
            for c in range(_NDC):
                sl = pl.ds(c * _L, _L)
                x = rows_v[r, sl]
                rows_v[r, sl] = (x - mu) * inv * gamma_v[sl] + beta_v[sl]
            return rcarry

        lax.fori_loop(0, _R, row_body, 0)
        pltpu.sync_copy(rows_v, out.at[pl.ds(g0, _R)])
        return carry

    lax.fori_loop(0, _NCHUNK, chunk_body, 0)


@jax.jit
def kernel(batched_tokens, batched_segments, tokens_table, segments_table,
           positions, gamma, beta):
    tok_idx = batched_tokens.reshape(_NW, _NCHUNK, _R)
    seg_idx = batched_segments.reshape(_NW, _NCHUNK, _R)
    mesh = plsc.VectorSubcoreMesh(core_axis_name="c", subcore_axis_name="s")
    fn = functools.partial(
        pl.kernel,
        mesh=mesh,
        out_type=jax.ShapeDtypeStruct((_B * _S, _D), jnp.float32),
        scratch_types=[
            pltpu.VMEM((_NCHUNK, _R), jnp.int32),
            pltpu.VMEM((_NCHUNK, _R), jnp.int32),
            pltpu.VMEM((_R, _D), jnp.float32),
            pltpu.VMEM((_R, _D), jnp.float32),
            pltpu.VMEM((_R, _D), jnp.float32),
            pltpu.VMEM((_D,), jnp.float32),
            pltpu.VMEM((_D,), jnp.float32),
            pltpu.SemaphoreType.DMA,
        ],
    )(_emb_body)
    out = fn(tok_idx, seg_idx, tokens_table, segments_table, positions,
             gamma, beta)
    return out.reshape(_B, _S, _D)


# speedup vs baseline: 3.2878x; 3.2878x over previous
import functools

import jax
import jax.numpy as jnp
from jax import lax
from jax.experimental import pallas as pl
from jax.experimental.pallas import tpu as pltpu
from jax.experimental.pallas import tpu_sc as plsc

_B, _S, _D = 4, 2048, 768
_NC, _NS = 2, 16
_NW = _NC * _NS
_RPW = (_B * _S) // _NW
_CH = 64
_NCHUNK = _RPW // _CH

_BR = 256
_NBLK = (_B * _S) // _BR
_PBLK = _S // _BR


def _gather_body(tok_idx, table, out, idx_v, buf0, buf1,
                 sem_in0, sem_in1, sem_out0, sem_out1):
    wid = lax.axis_index("s") * _NC + lax.axis_index("c")
    pltpu.sync_copy(tok_idx.at[wid], idx_v)
    g0 = wid * _RPW
    bufs = (buf0, buf1)
    sin = (sem_in0, sem_in1)
    sout = (sem_out0, sem_out1)

    def gin(ci, p):
        return pltpu.make_async_copy(table.at[idx_v.at[ci]], bufs[p], sin[p])

    def gout(ci, p):
        return pltpu.make_async_copy(
            bufs[p], out.at[pl.ds(g0 + ci * _CH, _CH)], sout[p])

    gin(0, 0).start()
    for ci in range(_NCHUNK):
        p = ci % 2
        if ci + 1 < _NCHUNK:
            if ci >= 1:
                gout(ci - 1, 1 - p).wait()
            gin(ci + 1, 1 - p).start()
        gin(ci, p).wait()
        gout(ci, p).start()
    gout(_NCHUNK - 2, _NCHUNK % 2).wait()
    gout(_NCHUNK - 1, 1 - _NCHUNK % 2).wait()


def _sc_gather(tok_idx, table):
    mesh = plsc.VectorSubcoreMesh(core_axis_name="c", subcore_axis_name="s")
    fn = functools.partial(
        pl.kernel,
        mesh=mesh,
        out_type=jax.ShapeDtypeStruct((_B * _S, _D), jnp.float32),
        scratch_types=[
            pltpu.VMEM((_NCHUNK, _CH), jnp.int32),
            pltpu.VMEM((_CH, _D), jnp.float32),
            pltpu.VMEM((_CH, _D), jnp.float32),
            pltpu.SemaphoreType.DMA,
            pltpu.SemaphoreType.DMA,
            pltpu.SemaphoreType.DMA,
            pltpu.SemaphoreType.DMA,
        ],
    )(_gather_body)
    return fn(tok_idx, table)


def _dense_body(seg_ref, st_ref, gam_ref, bet_ref, g_ref, pos_ref, o_ref):
    x = g_ref[...]
    seg = seg_ref[...]
    m0 = (seg == 0).astype(jnp.float32)
    m1 = (seg == 1).astype(jnp.float32)
    m2 = (seg == 2).astype(jnp.float32)
    x = (x + pos_ref[...]
         + m0 * st_ref[0:1, :] + m1 * st_ref[1:2, :] + m2 * st_ref[2:3, :])
    mu = jnp.mean(x, axis=1, keepdims=True)
    d = x - mu
    var = jnp.mean(d * d, axis=1, keepdims=True)
    inv = lax.rsqrt(var + jnp.float32(1e-5))
    o_ref[...] = d * inv * gam_ref[...] + bet_ref[...]


def _tc_dense(gathered, segments, seg_table, positions, gamma, beta):
    return pl.pallas_call(
        _dense_body,
        grid=(_NBLK,),
        in_specs=[
            pl.BlockSpec((_BR, 1), lambda i: (i, 0)),
            pl.BlockSpec((3, _D), lambda i: (0, 0)),
            pl.BlockSpec((1, _D), lambda i: (0, 0)),
            pl.BlockSpec((1, _D), lambda i: (0, 0)),
            pl.BlockSpec((_BR, _D), lambda i: (i, 0)),
            pl.BlockSpec((_BR, _D), lambda i: (i % _PBLK, 0)),
        ],
        out_specs=pl.BlockSpec((_BR, _D), lambda i: (i, 0)),
        out_shape=jax.ShapeDtypeStruct((_B * _S, _D), jnp.float32),
        compiler_params=pltpu.CompilerParams(
            dimension_semantics=("arbitrary",)),
    )(segments, seg_table, gamma, beta, gathered, positions)


@jax.jit
def kernel(batched_tokens, batched_segments, tokens_table, segments_table,
           positions, gamma, beta):
    tok_idx = batched_tokens.reshape(_NW, _NCHUNK, _CH)
    gathered = _sc_gather(tok_idx, tokens_table)
    out = _tc_dense(gathered, batched_segments.reshape(_B * _S, 1),
                    segments_table, positions.reshape(_S, _D),
                    gamma.reshape(1, _D), beta.reshape(1, _D))
    return out.reshape(_B, _S, _D)


# speedup vs baseline: 4.0945x; 1.2454x over previous
import functools

import jax
import jax.numpy as jnp
from jax import lax
from jax.experimental import pallas as pl
from jax.experimental.pallas import tpu as pltpu
from jax.experimental.pallas import tpu_sc as plsc

_B, _S, _D = 4, 2048, 768
_NC, _NS = 2, 16
_NW = _NC * _NS
_CH = 64

_NSPLIT = 2
_SS = _S // _NSPLIT
_BR = 256
_NBLK = _SS // _BR


def _gather_body(tok_idx, table, out, idx_v, buf0, buf1,
                 sem_in0, sem_in1, sem_out0, sem_out1):
    nchunk = idx_v.shape[0]
    rpw = nchunk * _CH
    wid = lax.axis_index("s") * _NC + lax.axis_index("c")
    pltpu.sync_copy(tok_idx.at[wid], idx_v)
    g0 = wid * rpw
    bufs = (buf0, buf1)
    sin = (sem_in0, sem_in1)
    sout = (sem_out0, sem_out1)

    def gin(ci, p):
        return pltpu.make_async_copy(table.at[idx_v.at[ci]], bufs[p], sin[p])

    def gout(ci, p):
        return pltpu.make_async_copy(
            bufs[p], out.at[pl.ds(g0 + ci * _CH, _CH)], sout[p])

    gin(0, 0).start()
    for ci in range(nchunk):
        p = ci % 2
        if ci + 1 < nchunk:
            if ci >= 1:
                gout(ci - 1, 1 - p).wait()
            gin(ci + 1, 1 - p).start()
        gin(ci, p).wait()
        gout(ci, p).start()
    for ci in range(max(0, nchunk - 2), nchunk):
        gout(ci, ci % 2).wait()


def _sc_gather(tok_idx, table):
    nchunk = tok_idx.shape[1]
    rows = _NW * nchunk * _CH
    mesh = plsc.VectorSubcoreMesh(core_axis_name="c", subcore_axis_name="s")
    fn = functools.partial(
        pl.kernel,
        mesh=mesh,
        out_type=jax.ShapeDtypeStruct((rows, _D), jnp.float32),
        scratch_types=[
            pltpu.VMEM((nchunk, _CH), jnp.int32),
            pltpu.VMEM((_CH, _D), jnp.float32),
            pltpu.VMEM((_CH, _D), jnp.float32),
            pltpu.SemaphoreType.DMA,
            pltpu.SemaphoreType.DMA,
            pltpu.SemaphoreType.DMA,
            pltpu.SemaphoreType.DMA,
        ],
    )(_gather_body)
    return fn(tok_idx, table)


def _dense_compute(seg_ref, st_ref, gam_ref, bet_ref, g_ref, pos_ref, o_ref):
    x = g_ref[...]
    seg = seg_ref[...]
    pos = pos_ref[...]
    p0 = pos + st_ref[0:1, 0:1, :]
    p1 = pos + st_ref[0:1, 1:2, :]
    x = x + jnp.where(seg == 0, p0, jnp.where(seg == 1, p1, pos))
    inv_d = jnp.float32(1.0 / _D)
    mu = jnp.sum(x, axis=2, keepdims=True) * inv_d
    s2 = jnp.sum(x * x, axis=2, keepdims=True) * inv_d
    var = s2 - mu * mu
    inv = lax.rsqrt(var + jnp.float32(1e-5))
    o_ref[...] = (x - mu) * inv * gam_ref[...] + bet_ref[...]


def _dense_body(seg_ref, st_ref, gam_ref, bet_ref, g_ref, pos_ref, o_ref):
    _dense_compute(seg_ref, st_ref, gam_ref, bet_ref, g_ref, pos_ref, o_ref)


def _dense_body_carry(seg_ref, st_ref, gam_ref, bet_ref, g_ref, pos_ref,
                      carry_ref, o_ref):
    del carry_ref
    _dense_compute(seg_ref, st_ref, gam_ref, bet_ref, g_ref, pos_ref, o_ref)


def _tc_dense(gathered, segments, seg_table, positions, gamma, beta, k,
              carry=None):
    in_specs = [
        pl.BlockSpec((_B, _BR, 1), lambda i: (0, k * _NBLK + i, 0)),
        pl.BlockSpec((1, 3, _D), lambda i: (0, 0, 0)),
        pl.BlockSpec((1, 1, _D), lambda i: (0, 0, 0)),
        pl.BlockSpec((1, 1, _D), lambda i: (0, 0, 0)),
        pl.BlockSpec((_B, _BR, _D), lambda i: (0, i, 0)),
        pl.BlockSpec((1, _BR, _D), lambda i: (0, k * _NBLK + i, 0)),
    ]
    args = [segments, seg_table, gamma, beta, gathered, positions]
    if carry is None:
        body = _dense_body
        io_alias = {}
    else:
        body = _dense_body_carry
        in_specs.append(pl.BlockSpec(memory_space=pl.ANY))
        args.append(carry)
        io_alias = {6: 0}
    return pl.pallas_call(
        body,
        grid=(_NBLK,),
        in_specs=in_specs,
        out_specs=pl.BlockSpec((_B, _BR, _D), lambda i: (0, k * _NBLK + i, 0)),
        out_shape=jax.ShapeDtypeStruct((_B, _S, _D), jnp.float32),
        input_output_aliases=io_alias,
        compiler_params=pltpu.CompilerParams(
            dimension_semantics=("arbitrary",)),
    )(*args)


@jax.jit
def kernel(batched_tokens, batched_segments, tokens_table, segments_table,
           positions, gamma, beta):
    seg3 = batched_segments.reshape(_B, _S, 1)
    st3 = segments_table.reshape(1, 3, _D)
    pos3 = positions.reshape(1, _S, _D)
    gam3 = gamma.reshape(1, 1, _D)
    bet3 = beta.reshape(1, 1, _D)

    gs = []
    for k in range(_NSPLIT):
        idx = lax.slice_in_dim(batched_tokens, k * _SS, (k + 1) * _SS, axis=1)
        gs.append(_sc_gather(idx.reshape(_NW, -1, _CH), tokens_table))

    out = None
    for k in range(_NSPLIT):
        out = _tc_dense(gs[k].reshape(_B, _SS, _D), seg3, st3, pos3,
                        gam3, bet3, k, carry=out)
    return out


# speedup vs baseline: 4.1441x; 1.0121x over previous
import functools

import jax
import jax.numpy as jnp
from jax import lax
from jax.experimental import pallas as pl
from jax.experimental.pallas import tpu as pltpu
from jax.experimental.pallas import tpu_sc as plsc

_B, _S, _D = 4, 2048, 768
_NC, _NS = 2, 16
_NW = _NC * _NS
_RPW = (_B * _S) // _NW
_CH = 64
_NCHUNK = _RPW // _CH

_BR = 512
_NBLK = _S // _BR


def _gather_body(tok_idx, table, out, idx_v, buf0, buf1,
                 sem_in0, sem_in1, sem_out0, sem_out1):
    wid = lax.axis_index("s") * _NC + lax.axis_index("c")
    pltpu.sync_copy(tok_idx.at[wid], idx_v)
    g0 = wid * _RPW
    bufs = (buf0, buf1)
    sin = (sem_in0, sem_in1)
    sout = (sem_out0, sem_out1)

    def gin(ci, p):
        return pltpu.make_async_copy(table.at[idx_v.at[ci]], bufs[p], sin[p])

    def gout(ci, p):
        return pltpu.make_async_copy(
            bufs[p], out.at[pl.ds(g0 + ci * _CH, _CH)], sout[p])

    gin(0, 0).start()
    for ci in range(_NCHUNK):
        p = ci % 2
        if ci + 1 < _NCHUNK:
            if ci >= 1:
                gout(ci - 1, 1 - p).wait()
            gin(ci + 1, 1 - p).start()
        gin(ci, p).wait()
        gout(ci, p).start()
    for ci in range(_NCHUNK - 2, _NCHUNK):
        gout(ci, ci % 2).wait()


def _sc_gather(tok_idx, table):
    mesh = plsc.VectorSubcoreMesh(core_axis_name="c", subcore_axis_name="s")
    fn = functools.partial(
        pl.kernel,
        mesh=mesh,
        out_type=jax.ShapeDtypeStruct((_B * _S, _D), jnp.float32),
        scratch_types=[
            pltpu.VMEM((_NCHUNK, _CH), jnp.int32),
            pltpu.VMEM((_CH, _D), jnp.float32),
            pltpu.VMEM((_CH, _D), jnp.float32),
            pltpu.SemaphoreType.DMA,
            pltpu.SemaphoreType.DMA,
            pltpu.SemaphoreType.DMA,
            pltpu.SemaphoreType.DMA,
        ],
    )(_gather_body)
    return fn(tok_idx, table)


def _dense_body(seg_ref, st_ref, gam_ref, bet_ref, g_ref, pos_ref, o_ref):
    x = g_ref[...]
    seg = seg_ref[...]
    pos = pos_ref[...]
    p0 = pos + st_ref[0:1, 0:1, :]
    p1 = pos + st_ref[0:1, 1:2, :]
    x = x + jnp.where(seg == 0, p0, jnp.where(seg == 1, p1, pos))
    inv_d = jnp.float32(1.0 / _D)
    mu = jnp.sum(x, axis=2, keepdims=True) * inv_d
    s2 = jnp.sum(x * x, axis=2, keepdims=True) * inv_d
    var = s2 - mu * mu
    inv = lax.rsqrt(var + jnp.float32(1e-5))
    o_ref[...] = (x - mu) * inv * gam_ref[...] + bet_ref[...]


def _tc_dense(gathered, segments, seg_table, positions, gamma, beta):
    return pl.pallas_call(
        _dense_body,
        grid=(_NBLK,),
        in_specs=[
            pl.BlockSpec((_B, _BR, 1), lambda i: (0, i, 0)),
            pl.BlockSpec((1, 3, _D), lambda i: (0, 0, 0)),
            pl.BlockSpec((1, 1, _D), lambda i: (0, 0, 0)),
            pl.BlockSpec((1, 1, _D), lambda i: (0, 0, 0)),
            pl.BlockSpec((_B, _BR, _D), lambda i: (0, i, 0)),
            pl.BlockSpec((1, _BR, _D), lambda i: (0, i, 0)),
        ],
        out_specs=pl.BlockSpec((_B, _BR, _D), lambda i: (0, i, 0)),
        out_shape=jax.ShapeDtypeStruct((_B, _S, _D), jnp.float32),
        compiler_params=pltpu.CompilerParams(
            dimension_semantics=("arbitrary",)),
    )(segments, seg_table, gamma, beta, gathered, positions)


@jax.jit
def kernel(batched_tokens, batched_segments, tokens_table, segments_table,
           positions, gamma, beta):
    tok_idx = batched_tokens.reshape(_NW, _NCHUNK, _CH)
    gathered = _sc_gather(tok_idx, tokens_table)
    return _tc_dense(gathered.reshape(_B, _S, _D),
                     batched_segments.reshape(_B, _S, 1),
                     segments_table.reshape(1, 3, _D),
                     positions.reshape(1, _S, _D),
                     gamma.reshape(1, 1, _D), beta.reshape(1, 1, _D))
